# Initial kernel scaffold; baseline (speedup 1.0000x reference)
import jax
import jax.numpy as jnp
from jax.experimental import pallas as pl


def kernel(h, e, edge_index, Wq, Wk, Wv, We):
    raise NotImplementedError("write your pallas kernel here")


# speedup vs baseline: 25.1202x; 25.1202x over previous
import functools

import jax
import jax.numpy as jnp
import numpy as np
from jax import lax
from jax.experimental import pallas as pl
from jax.experimental.pallas import tpu as pltpu
from jax.experimental.pallas import tpu_sc as plsc

N_NODES = 10000
N_EDGES = 320000
IN_DIM = 128
OUT_DIM = 16
NUM_HEADS = 8
D = NUM_HEADS * OUT_DIM

NC = 2
NS = 16
NW = NC * NS
EDGES_PER_W = N_EDGES // NW
CH = 80
N_CHUNKS = EDGES_PER_W // CH
ROWS_PER_TILE = 624
TAIL_ROWS = N_NODES - NS * ROWS_PER_TILE
TAIL_OFF = NS * ROWS_PER_TILE


def _proj_body(h_ref, wkv_ref, wq_ref, kv_ref, q_ref):
    hb = h_ref[...]
    kv_ref[...] = jnp.dot(hb, wkv_ref[...], preferred_element_type=jnp.float32)
    q_ref[...] = jnp.dot(hb, wq_ref[...], preferred_element_type=jnp.float32)


def _project(h, Wkv, Wq):
    blk = 1000
    grid = (N_NODES // blk,)
    return pl.pallas_call(
        _proj_body,
        grid=grid,
        in_specs=[
            pl.BlockSpec((blk, IN_DIM), lambda i: (i, 0)),
            pl.BlockSpec((IN_DIM, 2 * D), lambda i: (0, 0)),
            pl.BlockSpec((IN_DIM, D), lambda i: (0, 0)),
        ],
        out_specs=[
            pl.BlockSpec((blk, 2 * D), lambda i: (i, 0)),
            pl.BlockSpec((blk, D), lambda i: (i, 0)),
        ],
        out_shape=[
            jax.ShapeDtypeStruct((N_NODES, 2 * D), jnp.float32),
            jax.ShapeDtypeStruct((N_NODES, D), jnp.float32),
        ],
    )(h, Wkv, Wq)


def _gather_body(kv_hbm, q_hbm, src_hbm, dst_hbm, kvg_hbm, qg_hbm,
                 sidx, didx, kvrows, qrows, sem1, sem2):
    wid = lax.axis_index("s") * NC + lax.axis_index("c")
    base = wid * EDGES_PER_W

    def body(i, carry):
        off = base + i * CH
        pltpu.sync_copy(src_hbm.at[pl.ds(off, CH)], sidx)
        pltpu.sync_copy(dst_hbm.at[pl.ds(off, CH)], didx)
        ckv = pltpu.async_copy(kv_hbm.at[sidx], kvrows, sem1)
        cq = pltpu.async_copy(q_hbm.at[didx], qrows, sem2)
        ckv.wait()
        cq.wait()
        pltpu.sync_copy(kvrows, kvg_hbm.at[pl.ds(off, CH)])
        pltpu.sync_copy(qrows, qg_hbm.at[pl.ds(off, CH)])
        return carry

    lax.fori_loop(0, N_CHUNKS, body, 0)


def _gather(kv, q, src, dst):
    mesh = plsc.VectorSubcoreMesh(core_axis_name="c", subcore_axis_name="s")
    f = pl.kernel(
        _gather_body,
        out_type=[
            jax.ShapeDtypeStruct((N_EDGES, 2 * D), jnp.float32),
            jax.ShapeDtypeStruct((N_EDGES, D), jnp.float32),
        ],
        mesh=mesh,
        scratch_types=[
            pltpu.VMEM((CH,), jnp.int32),
            pltpu.VMEM((CH,), jnp.int32),
            pltpu.VMEM((CH, 2 * D), jnp.float32),
            pltpu.VMEM((CH, D), jnp.float32),
            pltpu.SemaphoreType.DMA,
            pltpu.SemaphoreType.DMA,
        ],
    )
    return f(kv, q, src, dst)


def _edge_body(e_ref, kvg_ref, qg_ref, we_ref, hm_ref, rm_ref,
               eout_ref, sv_ref, sx_ref):
    proj = jnp.dot(e_ref[...], we_ref[...], preferred_element_type=jnp.float32)
    kv = kvg_ref[...]
    k = kv[:, :D]
    v = kv[:, D:]
    score = k * qg_ref[...] * (1.0 / np.sqrt(OUT_DIM)) + proj
    eout_ref[...] = score
    hs = jnp.dot(score, hm_ref[...], preferred_element_type=jnp.float32,
                 precision=jax.lax.Precision.HIGHEST)
    s = jnp.exp(jnp.clip(hs, -5.0, 5.0))
    sexp = jnp.dot(s, rm_ref[...], preferred_element_type=jnp.float32,
                   precision=jax.lax.Precision.HIGHEST)
    sx_ref[...] = sexp
    sv_ref[...] = v * sexp


def _edge_stage(e, kvg, qg, We, Hmat, Rmat):
    blk = 4000
    grid = (N_EDGES // blk,)
    return pl.pallas_call(
        _edge_body,
        grid=grid,
        in_specs=[
            pl.BlockSpec((blk, IN_DIM), lambda i: (i, 0)),
            pl.BlockSpec((blk, 2 * D), lambda i: (i, 0)),
            pl.BlockSpec((blk, D), lambda i: (i, 0)),
            pl.BlockSpec((IN_DIM, D), lambda i: (0, 0)),
            pl.BlockSpec((D, 16), lambda i: (0, 0)),
            pl.BlockSpec((16, D), lambda i: (0, 0)),
        ],
        out_specs=[
            pl.BlockSpec((blk, D), lambda i: (i, 0)),
            pl.BlockSpec((blk, D), lambda i: (i, 0)),
            pl.BlockSpec((blk, D), lambda i: (i, 0)),
        ],
        out_shape=[
            jax.ShapeDtypeStruct((N_EDGES, D), jnp.float32),
            jax.ShapeDtypeStruct((N_EDGES, D), jnp.float32),
            jax.ShapeDtypeStruct((N_EDGES, D), jnp.float32),
        ],
    )(e, kvg, qg, We, Hmat, Rmat)


def _scatter_body(sv_hbm, sx_hbm, dst_hbm, zeros_hbm, wv_out, zx_out,
                  didx, rows, acc):
    cid = lax.axis_index("c")
    sid = lax.axis_index("s")
    wid = sid * NC + cid
    base = wid * EDGES_PER_W
    r0 = sid * ROWS_PER_TILE

    for src_hbm, out_hbm in ((sv_hbm, wv_out), (sx_hbm, zx_out)):
        pltpu.sync_copy(zeros_hbm.at[pl.ds(0, ROWS_PER_TILE)],
                        acc.at[pl.ds(r0, ROWS_PER_TILE)])

        @pl.when(sid == 0)
        def _zero_tail():
            pltpu.sync_copy(zeros_hbm.at[pl.ds(0, TAIL_ROWS)],
                            acc.at[pl.ds(TAIL_OFF, TAIL_ROWS)])

        plsc.subcore_barrier()

        def body(i, carry):
            off = base + i * CH
            pltpu.sync_copy(dst_hbm.at[pl.ds(off, CH)], didx)
            pltpu.sync_copy(src_hbm.at[pl.ds(off, CH)], rows)
            pltpu.sync_copy(rows, acc.at[didx], add=True)
            return carry

        lax.fori_loop(0, N_CHUNKS, body, 0)
        plsc.subcore_barrier()

        pltpu.sync_copy(acc.at[pl.ds(r0, ROWS_PER_TILE)],
                        out_hbm.at[cid, pl.ds(r0, ROWS_PER_TILE)])

        @pl.when(sid == 0)
        def _copy_tail():
            pltpu.sync_copy(acc.at[pl.ds(TAIL_OFF, TAIL_ROWS)],
                            out_hbm.at[cid, pl.ds(TAIL_OFF, TAIL_ROWS)])

        plsc.subcore_barrier()


def _scatter(sv, sx, dst, zeros):
    mesh = plsc.VectorSubcoreMesh(core_axis_name="c", subcore_axis_name="s")
    f = pl.kernel(
        _scatter_body,
        out_type=[
            jax.ShapeDtypeStruct((NC, N_NODES, D), jnp.float32),
            jax.ShapeDtypeStruct((NC, N_NODES, D), jnp.float32),
        ],
        mesh=mesh,
        scratch_types=[
            pltpu.VMEM((CH,), jnp.int32),
            pltpu.VMEM((CH, D), jnp.float32),
            pltpu.VMEM_SHARED((N_NODES, D), jnp.float32),
        ],
    )
    return f(sv, sx, dst, zeros)


def _combine_body(wv0_ref, wv1_ref, z0_ref, z1_ref, out_ref):
    wv = wv0_ref[...] + wv1_ref[...]
    zexp = z0_ref[...] + z1_ref[...]
    out_ref[...] = wv / (zexp + 1e-6)


def _combine(wv_part, zx_part):
    blk = 1000
    grid = (N_NODES // blk,)
    return pl.pallas_call(
        _combine_body,
        grid=grid,
        in_specs=[
            pl.BlockSpec((blk, D), lambda i: (i, 0)),
            pl.BlockSpec((blk, D), lambda i: (i, 0)),
            pl.BlockSpec((blk, D), lambda i: (i, 0)),
            pl.BlockSpec((blk, D), lambda i: (i, 0)),
        ],
        out_specs=pl.BlockSpec((blk, D), lambda i: (i, 0)),
        out_shape=jax.ShapeDtypeStruct((N_NODES, D), jnp.float32),
    )(wv_part[0], wv_part[1], zx_part[0], zx_part[1])


_HMAT = np.zeros((D, 16), np.float32)
for _h in range(NUM_HEADS):
    _HMAT[_h * OUT_DIM:(_h + 1) * OUT_DIM, _h] = 1.0
_RMAT = np.zeros((16, D), np.float32)
for _h in range(NUM_HEADS):
    _RMAT[_h, _h * OUT_DIM:(_h + 1) * OUT_DIM] = 1.0


def kernel(h, e, edge_index, Wq, Wk, Wv, We):
    src = edge_index[0]
    dst = edge_index[1]
    Wkv = jnp.concatenate([Wk, Wv], axis=1)
    Hmat = jnp.asarray(_HMAT)
    Rmat = jnp.asarray(_RMAT)
    zeros = jnp.zeros((ROWS_PER_TILE, D), jnp.float32)

    kv, q = _project(h, Wkv, Wq)
    kvg, qg = _gather(kv, q, src, dst)
    e_out, sv, sx = _edge_stage(e, kvg, qg, We, Hmat, Rmat)
    wv_part, zx_part = _scatter(sv, sx, dst, zeros)
    h_out = _combine(wv_part, zx_part)
    return h_out, e_out


# speedup vs baseline: 31.3689x; 1.2488x over previous
import functools

import jax
import jax.numpy as jnp
import numpy as np
from jax import lax
from jax.experimental import pallas as pl
from jax.experimental.pallas import tpu as pltpu
from jax.experimental.pallas import tpu_sc as plsc

N_NODES = 10000
N_EDGES = 320000
IN_DIM = 128
OUT_DIM = 16
NUM_HEADS = 8
D = NUM_HEADS * OUT_DIM

NC = 2
NS = 16
NW = NC * NS
EDGES_PER_W = N_EDGES // NW
CH = 80
N_CHUNKS = EDGES_PER_W // CH
ROWS_PER_TILE = 624
TAIL_ROWS = N_NODES - NS * ROWS_PER_TILE
TAIL_OFF = NS * ROWS_PER_TILE


def _proj_body(h_ref, wkv_ref, wq_ref, kv_ref, q_ref):
    hb = h_ref[...]
    kv_ref[...] = jnp.dot(hb, wkv_ref[...], preferred_element_type=jnp.float32)
    q_ref[...] = jnp.dot(hb, wq_ref[...], preferred_element_type=jnp.float32)


def _project(h, Wkv, Wq):
    blk = 1000
    grid = (N_NODES // blk,)
    return pl.pallas_call(
        _proj_body,
        grid=grid,
        in_specs=[
            pl.BlockSpec((blk, IN_DIM), lambda i: (i, 0)),
            pl.BlockSpec((IN_DIM, 2 * D), lambda i: (0, 0)),
            pl.BlockSpec((IN_DIM, D), lambda i: (0, 0)),
        ],
        out_specs=[
            pl.BlockSpec((blk, 2 * D), lambda i: (i, 0)),
            pl.BlockSpec((blk, D), lambda i: (i, 0)),
        ],
        out_shape=[
            jax.ShapeDtypeStruct((N_NODES, 2 * D), jnp.float32),
            jax.ShapeDtypeStruct((N_NODES, D), jnp.float32),
        ],
    )(h, Wkv, Wq)


def _gather_body(kv_hbm, q_hbm, src_hbm, dst_hbm, kvg_hbm, qg_hbm,
                 sidx, didx, kvrows, qrows, gsem, wsem):
    wid = lax.axis_index("s") * NC + lax.axis_index("c")
    base = wid * EDGES_PER_W

    def issue(i, b):
        off = base + i * CH
        pltpu.sync_copy(src_hbm.at[pl.ds(off, CH)], sidx[b])
        pltpu.sync_copy(dst_hbm.at[pl.ds(off, CH)], didx[b])
        pltpu.async_copy(kv_hbm.at[sidx[b]], kvrows[b], gsem[b])
        pltpu.async_copy(q_hbm.at[didx[b]], qrows[b], gsem[b])

    def wait_gathers(b):
        pltpu.make_async_copy(kv_hbm.at[sidx[b]], kvrows[b], gsem[b]).wait()
        pltpu.make_async_copy(q_hbm.at[didx[b]], qrows[b], gsem[b]).wait()

    def start_writes(i, b):
        off = base + i * CH
        pltpu.async_copy(kvrows[b], kvg_hbm.at[pl.ds(off, CH)], wsem[b])
        pltpu.async_copy(qrows[b], qg_hbm.at[pl.ds(off, CH)], wsem[b])

    def wait_writes(i, b):
        off = base + i * CH
        pltpu.make_async_copy(kvrows[b], kvg_hbm.at[pl.ds(off, CH)], wsem[b]).wait()
        pltpu.make_async_copy(qrows[b], qg_hbm.at[pl.ds(off, CH)], wsem[b]).wait()

    issue(0, 0)
    issue(1, 1)
    wait_gathers(0)
    start_writes(0, 0)

    def pbody(p, carry):
        j0 = 2 * p + 1
        wait_writes(j0 - 1, 0)
        issue(j0 + 1, 0)
        wait_gathers(1)
        start_writes(j0, 1)
        wait_writes(j0, 1)
        issue(j0 + 2, 1)
        wait_gathers(0)
        start_writes(j0 + 1, 0)
        return carry

    lax.fori_loop(0, (N_CHUNKS - 3) // 2, pbody, 0)
    wait_writes(122, 0)
    issue(124, 0)
    wait_gathers(1)
    start_writes(123, 1)
    wait_gathers(0)
    start_writes(124, 0)
    wait_writes(123, 1)
    wait_writes(124, 0)


def _gather(kv, q, src, dst):
    mesh = plsc.VectorSubcoreMesh(core_axis_name="c", subcore_axis_name="s")
    f = pl.kernel(
        _gather_body,
        out_type=[
            jax.ShapeDtypeStruct((N_EDGES, 2 * D), jnp.float32),
            jax.ShapeDtypeStruct((N_EDGES, D), jnp.float32),
        ],
        mesh=mesh,
        scratch_types=[
            [pltpu.VMEM((CH,), jnp.int32)] * 2,
            [pltpu.VMEM((CH,), jnp.int32)] * 2,
            [pltpu.VMEM((CH, 2 * D), jnp.float32)] * 2,
            [pltpu.VMEM((CH, D), jnp.float32)] * 2,
            [pltpu.SemaphoreType.DMA] * 2,
            [pltpu.SemaphoreType.DMA] * 2,
        ],
    )
    return f(kv, q, src, dst)


def _edge_body(e_ref, kvg_ref, qg_ref, we_ref, hm_ref, rm_ref,
               eout_ref, sv_ref, sx_ref):
    proj = jnp.dot(e_ref[...], we_ref[...], preferred_element_type=jnp.float32)
    kv = kvg_ref[...]
    k = kv[:, :D]
    v = kv[:, D:]
    score = k * qg_ref[...] * (1.0 / np.sqrt(OUT_DIM)) + proj
    eout_ref[...] = score
    hs = jnp.dot(score, hm_ref[...], preferred_element_type=jnp.float32,
                 precision=jax.lax.Precision.HIGHEST)
    s = jnp.exp(jnp.clip(hs, -5.0, 5.0))
    sexp = jnp.dot(s, rm_ref[...], preferred_element_type=jnp.float32,
                   precision=jax.lax.Precision.HIGHEST)
    sx_ref[...] = sexp
    sv_ref[...] = v * sexp


def _edge_stage(e, kvg, qg, We, Hmat, Rmat):
    blk = 2560
    grid = (N_EDGES // blk,)
    return pl.pallas_call(
        _edge_body,
        grid=grid,
        in_specs=[
            pl.BlockSpec((blk, IN_DIM), lambda i: (i, 0)),
            pl.BlockSpec((blk, 2 * D), lambda i: (i, 0)),
            pl.BlockSpec((blk, D), lambda i: (i, 0)),
            pl.BlockSpec((IN_DIM, D), lambda i: (0, 0)),
            pl.BlockSpec((D, 16), lambda i: (0, 0)),
            pl.BlockSpec((16, D), lambda i: (0, 0)),
        ],
        out_specs=[
            pl.BlockSpec((blk, D), lambda i: (i, 0)),
            pl.BlockSpec((blk, D), lambda i: (i, 0)),
            pl.BlockSpec((blk, D), lambda i: (i, 0)),
        ],
        out_shape=[
            jax.ShapeDtypeStruct((N_EDGES, D), jnp.float32),
            jax.ShapeDtypeStruct((N_EDGES, D), jnp.float32),
            jax.ShapeDtypeStruct((N_EDGES, D), jnp.float32),
        ],
    )(e, kvg, qg, We, Hmat, Rmat)


def _scatter_body(sv_hbm, sx_hbm, dst_hbm, zeros_hbm, wv_out, zx_out,
                  didx, rows, lsem, ssem, acc):
    cid = lax.axis_index("c")
    sid = lax.axis_index("s")
    wid = sid * NC + cid
    base = wid * EDGES_PER_W
    r0 = sid * ROWS_PER_TILE

    for src_hbm, out_hbm in ((sv_hbm, wv_out), (sx_hbm, zx_out)):
        pltpu.sync_copy(zeros_hbm.at[pl.ds(0, ROWS_PER_TILE)],
                        acc.at[pl.ds(r0, ROWS_PER_TILE)])

        @pl.when(sid == 0)
        def _zero_tail():
            pltpu.sync_copy(zeros_hbm.at[pl.ds(0, TAIL_ROWS)],
                            acc.at[pl.ds(TAIL_OFF, TAIL_ROWS)])

        plsc.subcore_barrier()

        def load(i, b):
            off = base + i * CH
            pltpu.async_copy(dst_hbm.at[pl.ds(off, CH)], didx[b], lsem[b])
            pltpu.async_copy(src_hbm.at[pl.ds(off, CH)], rows[b], lsem[b])

        def wait_load(i, b):
            off = base + i * CH
            pltpu.make_async_copy(dst_hbm.at[pl.ds(off, CH)], didx[b],
                                  lsem[b]).wait()
            pltpu.make_async_copy(src_hbm.at[pl.ds(off, CH)], rows[b],
                                  lsem[b]).wait()

        load(0, 0)

        def pbody(p, carry):
            j0 = 2 * p
            wait_load(j0, 0)
            load(j0 + 1, 1)
            pltpu.sync_copy(rows[0], acc.at[didx[0]], add=True)
            wait_load(j0 + 1, 1)
            load(j0 + 2, 0)
            pltpu.sync_copy(rows[1], acc.at[didx[1]], add=True)
            return carry

        lax.fori_loop(0, (N_CHUNKS - 1) // 2, pbody, 0)
        wait_load(N_CHUNKS - 1, 0)
        pltpu.sync_copy(rows[0], acc.at[didx[0]], add=True)
        plsc.subcore_barrier()

        pltpu.sync_copy(acc.at[pl.ds(r0, ROWS_PER_TILE)],
                        out_hbm.at[cid, pl.ds(r0, ROWS_PER_TILE)])

        @pl.when(sid == 0)
        def _copy_tail():
            pltpu.sync_copy(acc.at[pl.ds(TAIL_OFF, TAIL_ROWS)],
                            out_hbm.at[cid, pl.ds(TAIL_OFF, TAIL_ROWS)])

        plsc.subcore_barrier()


def _scatter(sv, sx, dst, zeros):
    mesh = plsc.VectorSubcoreMesh(core_axis_name="c", subcore_axis_name="s")
    f = pl.kernel(
        _scatter_body,
        out_type=[
            jax.ShapeDtypeStruct((NC, N_NODES, D), jnp.float32),
            jax.ShapeDtypeStruct((NC, N_NODES, D), jnp.float32),
        ],
        mesh=mesh,
        scratch_types=[
            [pltpu.VMEM((CH,), jnp.int32)] * 2,
            [pltpu.VMEM((CH, D), jnp.float32)] * 2,
            [pltpu.SemaphoreType.DMA] * 2,
            [pltpu.SemaphoreType.DMA] * 2,
            pltpu.VMEM_SHARED((N_NODES, D), jnp.float32),
        ],
    )
    return f(sv, sx, dst, zeros)


def _combine_body(wv0_ref, wv1_ref, z0_ref, z1_ref, out_ref):
    wv = wv0_ref[...] + wv1_ref[...]
    zexp = z0_ref[...] + z1_ref[...]
    out_ref[...] = wv / (zexp + 1e-6)


def _combine(wv_part, zx_part):
    blk = 1000
    grid = (N_NODES // blk,)
    return pl.pallas_call(
        _combine_body,
        grid=grid,
        in_specs=[
            pl.BlockSpec((blk, D), lambda i: (i, 0)),
            pl.BlockSpec((blk, D), lambda i: (i, 0)),
            pl.BlockSpec((blk, D), lambda i: (i, 0)),
            pl.BlockSpec((blk, D), lambda i: (i, 0)),
        ],
        out_specs=pl.BlockSpec((blk, D), lambda i: (i, 0)),
        out_shape=jax.ShapeDtypeStruct((N_NODES, D), jnp.float32),
    )(wv_part[0], wv_part[1], zx_part[0], zx_part[1])


_HMAT = np.zeros((D, 16), np.float32)
for _h in range(NUM_HEADS):
    _HMAT[_h * OUT_DIM:(_h + 1) * OUT_DIM, _h] = 1.0
_RMAT = np.zeros((16, D), np.float32)
for _h in range(NUM_HEADS):
    _RMAT[_h, _h * OUT_DIM:(_h + 1) * OUT_DIM] = 1.0


def kernel(h, e, edge_index, Wq, Wk, Wv, We):
    src = edge_index[0]
    dst = edge_index[1]
    Wkv = jnp.concatenate([Wk, Wv], axis=1)
    Hmat = jnp.asarray(_HMAT)
    Rmat = jnp.asarray(_RMAT)
    zeros = jnp.zeros((ROWS_PER_TILE, D), jnp.float32)

    kv, q = _project(h, Wkv, Wq)
    kvg, qg = _gather(kv, q, src, dst)
    e_out, sv, sx = _edge_stage(e, kvg, qg, We, Hmat, Rmat)
    wv_part, zx_part = _scatter(sv, sx, dst, zeros)
    h_out = _combine(wv_part, zx_part)
    return h_out, e_out


# speedup vs baseline: 32.6804x; 1.0418x over previous
import functools

import jax
import jax.numpy as jnp
import numpy as np
from jax import lax
from jax.experimental import pallas as pl
from jax.experimental.pallas import tpu as pltpu
from jax.experimental.pallas import tpu_sc as plsc

N_NODES = 10000
N_EDGES = 320000
IN_DIM = 128
OUT_DIM = 16
NUM_HEADS = 8
D = NUM_HEADS * OUT_DIM

NC = 2
NS = 16
NW = NC * NS
EDGES_PER_W = N_EDGES // NW
CH = 80
N_CHUNKS = EDGES_PER_W // CH
ROWS_PER_TILE = 624
TAIL_ROWS = N_NODES - NS * ROWS_PER_TILE
TAIL_OFF = NS * ROWS_PER_TILE


def _proj_body(h_ref, wkv_ref, wq_ref, kv_ref, q_ref):
    hb = h_ref[...]
    kv_ref[...] = jnp.dot(hb, wkv_ref[...], preferred_element_type=jnp.float32)
    q_ref[...] = jnp.dot(hb, wq_ref[...], preferred_element_type=jnp.float32)


def _project(h, Wkv, Wq):
    blk = 1000
    grid = (N_NODES // blk,)
    return pl.pallas_call(
        _proj_body,
        grid=grid,
        in_specs=[
            pl.BlockSpec((blk, IN_DIM), lambda i: (i, 0)),
            pl.BlockSpec((IN_DIM, 2 * D), lambda i: (0, 0)),
            pl.BlockSpec((IN_DIM, D), lambda i: (0, 0)),
        ],
        out_specs=[
            pl.BlockSpec((blk, 2 * D), lambda i: (i, 0)),
            pl.BlockSpec((blk, D), lambda i: (i, 0)),
        ],
        out_shape=[
            jax.ShapeDtypeStruct((N_NODES, 2 * D), jnp.float32),
            jax.ShapeDtypeStruct((N_NODES, D), jnp.float32),
        ],
    )(h, Wkv, Wq)


def _gather_body(kv_hbm, q_hbm, src_hbm, dst_hbm, kvg_hbm, qg_hbm,
                 sidx_all, didx_all, kvrows, qrows, gsem, wsem):
    wid = lax.axis_index("s") * NC + lax.axis_index("c")
    base = wid * EDGES_PER_W

    pltpu.sync_copy(src_hbm.at[pl.ds(base, EDGES_PER_W)], sidx_all)
    pltpu.sync_copy(dst_hbm.at[pl.ds(base, EDGES_PER_W)], didx_all)

    def issue(i, b):
        loff = i * CH
        pltpu.async_copy(kv_hbm.at[sidx_all.at[pl.ds(loff, CH)]],
                         kvrows[b], gsem[b])
        pltpu.async_copy(q_hbm.at[didx_all.at[pl.ds(loff, CH)]],
                         qrows[b], gsem[b])

    def wait_gathers(b):
        pltpu.make_async_copy(kv_hbm.at[sidx_all.at[pl.ds(0, CH)]],
                              kvrows[b], gsem[b]).wait()
        pltpu.make_async_copy(q_hbm.at[didx_all.at[pl.ds(0, CH)]],
                              qrows[b], gsem[b]).wait()

    def start_writes(i, b):
        off = base + i * CH
        pltpu.async_copy(kvrows[b], kvg_hbm.at[pl.ds(off, CH)], wsem[b])
        pltpu.async_copy(qrows[b], qg_hbm.at[pl.ds(off, CH)], wsem[b])

    def wait_writes(i, b):
        off = base + i * CH
        pltpu.make_async_copy(kvrows[b], kvg_hbm.at[pl.ds(off, CH)], wsem[b]).wait()
        pltpu.make_async_copy(qrows[b], qg_hbm.at[pl.ds(off, CH)], wsem[b]).wait()

    issue(0, 0)
    issue(1, 1)
    wait_gathers(0)
    start_writes(0, 0)

    def pbody(p, carry):
        j0 = 2 * p + 1
        wait_writes(j0 - 1, 0)
        issue(j0 + 1, 0)
        wait_gathers(1)
        start_writes(j0, 1)
        wait_writes(j0, 1)
        issue(j0 + 2, 1)
        wait_gathers(0)
        start_writes(j0 + 1, 0)
        return carry

    lax.fori_loop(0, (N_CHUNKS - 3) // 2, pbody, 0)
    wait_writes(122, 0)
    issue(124, 0)
    wait_gathers(1)
    start_writes(123, 1)
    wait_gathers(0)
    start_writes(124, 0)
    wait_writes(123, 1)
    wait_writes(124, 0)


def _gather(kv, q, src, dst):
    mesh = plsc.VectorSubcoreMesh(core_axis_name="c", subcore_axis_name="s")
    f = pl.kernel(
        _gather_body,
        out_type=[
            jax.ShapeDtypeStruct((N_EDGES, 2 * D), jnp.float32),
            jax.ShapeDtypeStruct((N_EDGES, D), jnp.float32),
        ],
        mesh=mesh,
        scratch_types=[
            pltpu.VMEM((EDGES_PER_W,), jnp.int32),
            pltpu.VMEM((EDGES_PER_W,), jnp.int32),
            [pltpu.VMEM((CH, 2 * D), jnp.float32)] * 2,
            [pltpu.VMEM((CH, D), jnp.float32)] * 2,
            [pltpu.SemaphoreType.DMA] * 2,
            [pltpu.SemaphoreType.DMA] * 2,
        ],
    )
    return f(kv, q, src, dst)


def _edge_body(e_ref, kvg_ref, qg_ref, we_ref, hm_ref, rm_ref,
               eout_ref, sv_ref, sx_ref):
    proj = jnp.dot(e_ref[...], we_ref[...], preferred_element_type=jnp.float32)
    kv = kvg_ref[...]
    k = kv[:, :D]
    v = kv[:, D:]
    score = k * qg_ref[...] * (1.0 / np.sqrt(OUT_DIM)) + proj
    eout_ref[...] = score
    hs = jnp.dot(score, hm_ref[...], preferred_element_type=jnp.float32,
                 precision=jax.lax.Precision.HIGHEST)
    s = jnp.exp(jnp.clip(hs, -5.0, 5.0))
    sexp = jnp.dot(s, rm_ref[...], preferred_element_type=jnp.float32,
                   precision=jax.lax.Precision.HIGHEST)
    sx_ref[...] = sexp
    sv_ref[...] = v * sexp


def _edge_stage(e, kvg, qg, We, Hmat, Rmat):
    blk = 2560
    grid = (N_EDGES // blk,)
    return pl.pallas_call(
        _edge_body,
        grid=grid,
        in_specs=[
            pl.BlockSpec((blk, IN_DIM), lambda i: (i, 0)),
            pl.BlockSpec((blk, 2 * D), lambda i: (i, 0)),
            pl.BlockSpec((blk, D), lambda i: (i, 0)),
            pl.BlockSpec((IN_DIM, D), lambda i: (0, 0)),
            pl.BlockSpec((D, 16), lambda i: (0, 0)),
            pl.BlockSpec((16, D), lambda i: (0, 0)),
        ],
        out_specs=[
            pl.BlockSpec((blk, D), lambda i: (i, 0)),
            pl.BlockSpec((blk, D), lambda i: (i, 0)),
            pl.BlockSpec((blk, D), lambda i: (i, 0)),
        ],
        out_shape=[
            jax.ShapeDtypeStruct((N_EDGES, D), jnp.float32),
            jax.ShapeDtypeStruct((N_EDGES, D), jnp.float32),
            jax.ShapeDtypeStruct((N_EDGES, D), jnp.float32),
        ],
    )(e, kvg, qg, We, Hmat, Rmat)


def _scatter_body(sv_hbm, sx_hbm, dst3_hbm, zeros_hbm, wv_out, zx_out,
                  didx2, rows, lsem, ssem, acc):
    cid = lax.axis_index("c")
    sid = lax.axis_index("s")
    wid = sid * NC + cid
    base = wid * EDGES_PER_W
    r0 = sid * ROWS_PER_TILE

    pltpu.sync_copy(dst3_hbm.at[wid], didx2)

    for src_hbm, out_hbm in ((sv_hbm, wv_out), (sx_hbm, zx_out)):
        pltpu.sync_copy(zeros_hbm.at[pl.ds(0, ROWS_PER_TILE)],
                        acc.at[pl.ds(r0, ROWS_PER_TILE)])

        @pl.when(sid == 0)
        def _zero_tail():
            pltpu.sync_copy(zeros_hbm.at[pl.ds(0, TAIL_ROWS)],
                            acc.at[pl.ds(TAIL_OFF, TAIL_ROWS)])

        plsc.subcore_barrier()

        def load(i, b):
            off = base + i * CH
            pltpu.async_copy(src_hbm.at[pl.ds(off, CH)], rows[b], lsem[b])

        def wait_load(i, b):
            off = base + i * CH
            pltpu.make_async_copy(src_hbm.at[pl.ds(off, CH)], rows[b],
                                  lsem[b]).wait()

        load(0, 0)

        def pbody(p, carry):
            j0 = 2 * p
            wait_load(j0, 0)
            load(j0 + 1, 1)
            pltpu.sync_copy(rows[0], acc.at[didx2.at[j0]], add=True)
            wait_load(j0 + 1, 1)
            load(j0 + 2, 0)
            pltpu.sync_copy(rows[1], acc.at[didx2.at[j0 + 1]], add=True)
            return carry

        lax.fori_loop(0, (N_CHUNKS - 1) // 2, pbody, 0)
        wait_load(N_CHUNKS - 1, 0)
        pltpu.sync_copy(rows[0], acc.at[didx2.at[N_CHUNKS - 1]], add=True)
        plsc.subcore_barrier()

        pltpu.sync_copy(acc.at[pl.ds(r0, ROWS_PER_TILE)],
                        out_hbm.at[cid, pl.ds(r0, ROWS_PER_TILE)])

        @pl.when(sid == 0)
        def _copy_tail():
            pltpu.sync_copy(acc.at[pl.ds(TAIL_OFF, TAIL_ROWS)],
                            out_hbm.at[cid, pl.ds(TAIL_OFF, TAIL_ROWS)])

        plsc.subcore_barrier()


def _scatter(sv, sx, dst, zeros):
    mesh = plsc.VectorSubcoreMesh(core_axis_name="c", subcore_axis_name="s")
    f = pl.kernel(
        _scatter_body,
        out_type=[
            jax.ShapeDtypeStruct((NC, N_NODES, D), jnp.float32),
            jax.ShapeDtypeStruct((NC, N_NODES, D), jnp.float32),
        ],
        mesh=mesh,
        scratch_types=[
            pltpu.VMEM((N_CHUNKS, CH), jnp.int32),
            [pltpu.VMEM((CH, D), jnp.float32)] * 2,
            [pltpu.SemaphoreType.DMA] * 2,
            [pltpu.SemaphoreType.DMA] * 2,
            pltpu.VMEM_SHARED((N_NODES, D), jnp.float32),
        ],
    )
    return f(sv, sx, dst.reshape(NW, N_CHUNKS, CH), zeros)


def _combine_body(wv0_ref, wv1_ref, z0_ref, z1_ref, out_ref):
    wv = wv0_ref[...] + wv1_ref[...]
    zexp = z0_ref[...] + z1_ref[...]
    out_ref[...] = wv / (zexp + 1e-6)


def _combine(wv_part, zx_part):
    blk = 1000
    grid = (N_NODES // blk,)
    return pl.pallas_call(
        _combine_body,
        grid=grid,
        in_specs=[
            pl.BlockSpec((blk, D), lambda i: (i, 0)),
            pl.BlockSpec((blk, D), lambda i: (i, 0)),
            pl.BlockSpec((blk, D), lambda i: (i, 0)),
            pl.BlockSpec((blk, D), lambda i: (i, 0)),
        ],
        out_specs=pl.BlockSpec((blk, D), lambda i: (i, 0)),
        out_shape=jax.ShapeDtypeStruct((N_NODES, D), jnp.float32),
    )(wv_part[0], wv_part[1], zx_part[0], zx_part[1])


_HMAT = np.zeros((D, 16), np.float32)
for _h in range(NUM_HEADS):
    _HMAT[_h * OUT_DIM:(_h + 1) * OUT_DIM, _h] = 1.0
_RMAT = np.zeros((16, D), np.float32)
for _h in range(NUM_HEADS):
    _RMAT[_h, _h * OUT_DIM:(_h + 1) * OUT_DIM] = 1.0


def kernel(h, e, edge_index, Wq, Wk, Wv, We):
    src = edge_index[0]
    dst = edge_index[1]
    Wkv = jnp.concatenate([Wk, Wv], axis=1)
    Hmat = jnp.asarray(_HMAT)
    Rmat = jnp.asarray(_RMAT)
    zeros = jnp.zeros((ROWS_PER_TILE, D), jnp.float32)

    kv, q = _project(h, Wkv, Wq)
    kvg, qg = _gather(kv, q, src, dst)
    e_out, sv, sx = _edge_stage(e, kvg, qg, We, Hmat, Rmat)
    wv_part, zx_part = _scatter(sv, sx, dst, zeros)
    h_out = _combine(wv_part, zx_part)
    return h_out, e_out


# speedup vs baseline: 33.3469x; 1.0204x over previous
import functools

import jax
import jax.numpy as jnp
import numpy as np
from jax import lax
from jax.experimental import pallas as pl
from jax.experimental.pallas import tpu as pltpu
from jax.experimental.pallas import tpu_sc as plsc

N_NODES = 10000
N_EDGES = 320000
IN_DIM = 128
OUT_DIM = 16
NUM_HEADS = 8
D = NUM_HEADS * OUT_DIM

NC = 2
NS = 16
NW = NC * NS
EDGES_PER_W = N_EDGES // NW
CH = 80
N_CHUNKS = EDGES_PER_W // CH
ROWS_PER_TILE = 624
TAIL_ROWS = N_NODES - NS * ROWS_PER_TILE
TAIL_OFF = NS * ROWS_PER_TILE


def _proj_body(h_ref, wkv_ref, wq_ref, kv_ref, q_ref):
    hb = h_ref[...]
    kv_ref[...] = jnp.dot(hb, wkv_ref[...], preferred_element_type=jnp.float32)
    q_ref[...] = jnp.dot(hb, wq_ref[...], preferred_element_type=jnp.float32)


def _project(h, Wkv, Wq):
    blk = 1000
    grid = (N_NODES // blk,)
    return pl.pallas_call(
        _proj_body,
        grid=grid,
        in_specs=[
            pl.BlockSpec((blk, IN_DIM), lambda i: (i, 0)),
            pl.BlockSpec((IN_DIM, 2 * D), lambda i: (0, 0)),
            pl.BlockSpec((IN_DIM, D), lambda i: (0, 0)),
        ],
        out_specs=[
            pl.BlockSpec((blk, 2 * D), lambda i: (i, 0)),
            pl.BlockSpec((blk, D), lambda i: (i, 0)),
        ],
        out_shape=[
            jax.ShapeDtypeStruct((N_NODES, 2 * D), jnp.float32),
            jax.ShapeDtypeStruct((N_NODES, D), jnp.float32),
        ],
    )(h, Wkv, Wq)


def _gather_body(kv_hbm, q_hbm, src_hbm, dst_hbm, kvg_hbm, qg_hbm,
                 sidx_all, didx_all, kvrows, qrows, gsem, wsem):
    wid = lax.axis_index("s") * NC + lax.axis_index("c")
    base = wid * EDGES_PER_W

    pltpu.sync_copy(src_hbm.at[pl.ds(base, EDGES_PER_W)], sidx_all)
    pltpu.sync_copy(dst_hbm.at[pl.ds(base, EDGES_PER_W)], didx_all)

    def issue(i, b):
        loff = i * CH
        pltpu.async_copy(kv_hbm.at[sidx_all.at[pl.ds(loff, CH)]],
                         kvrows[b], gsem[b])
        pltpu.async_copy(q_hbm.at[didx_all.at[pl.ds(loff, CH)]],
                         qrows[b], gsem[b])

    def wait_gathers(b):
        pltpu.make_async_copy(kv_hbm.at[sidx_all.at[pl.ds(0, CH)]],
                              kvrows[b], gsem[b]).wait()
        pltpu.make_async_copy(q_hbm.at[didx_all.at[pl.ds(0, CH)]],
                              qrows[b], gsem[b]).wait()

    def start_writes(i, b):
        off = base + i * CH
        pltpu.async_copy(kvrows[b], kvg_hbm.at[pl.ds(off, CH)], wsem[b])
        pltpu.async_copy(qrows[b], qg_hbm.at[pl.ds(off, CH)], wsem[b])

    def wait_writes(i, b):
        off = base + i * CH
        pltpu.make_async_copy(kvrows[b], kvg_hbm.at[pl.ds(off, CH)], wsem[b]).wait()
        pltpu.make_async_copy(qrows[b], qg_hbm.at[pl.ds(off, CH)], wsem[b]).wait()

    issue(0, 0)
    issue(1, 1)
    wait_gathers(0)
    start_writes(0, 0)

    def pbody(p, carry):
        j0 = 2 * p + 1
        wait_writes(j0 - 1, 0)
        issue(j0 + 1, 0)
        wait_gathers(1)
        start_writes(j0, 1)
        wait_writes(j0, 1)
        issue(j0 + 2, 1)
        wait_gathers(0)
        start_writes(j0 + 1, 0)
        return carry

    lax.fori_loop(0, (N_CHUNKS - 3) // 2, pbody, 0)
    wait_writes(122, 0)
    issue(124, 0)
    wait_gathers(1)
    start_writes(123, 1)
    wait_gathers(0)
    start_writes(124, 0)
    wait_writes(123, 1)
    wait_writes(124, 0)


def _gather(kv, q, src, dst):
    mesh = plsc.VectorSubcoreMesh(core_axis_name="c", subcore_axis_name="s")
    f = pl.kernel(
        _gather_body,
        out_type=[
            jax.ShapeDtypeStruct((N_EDGES, 2 * D), jnp.float32),
            jax.ShapeDtypeStruct((N_EDGES, D), jnp.float32),
        ],
        mesh=mesh,
        scratch_types=[
            pltpu.VMEM((EDGES_PER_W,), jnp.int32),
            pltpu.VMEM((EDGES_PER_W,), jnp.int32),
            [pltpu.VMEM((CH, 2 * D), jnp.float32)] * 2,
            [pltpu.VMEM((CH, D), jnp.float32)] * 2,
            [pltpu.SemaphoreType.DMA] * 2,
            [pltpu.SemaphoreType.DMA] * 2,
        ],
    )
    return f(kv, q, src, dst)


def _edge_body(e_ref, kvg_ref, qg_ref, we_ref, hm_ref, rm_ref,
               eout_ref, sv_ref, sx_ref):
    proj = jnp.dot(e_ref[...], we_ref[...], preferred_element_type=jnp.float32)
    kv = kvg_ref[...]
    k = kv[:, :D]
    v = kv[:, D:]
    score = k * qg_ref[...] * (1.0 / np.sqrt(OUT_DIM)) + proj
    eout_ref[...] = score
    hs = jnp.dot(score, hm_ref[...], preferred_element_type=jnp.float32,
                 precision=jax.lax.Precision.HIGHEST)
    s = jnp.exp(jnp.clip(hs, -5.0, 5.0))
    sexp = jnp.dot(s, rm_ref[...], preferred_element_type=jnp.float32,
                   precision=jax.lax.Precision.HIGHEST)
    sx_ref[...] = sexp
    sv_ref[...] = v * sexp


def _edge_stage(e, kvg, qg, We, Hmat, Rmat):
    blk = 5120
    grid = (N_EDGES // blk,)
    return pl.pallas_call(
        _edge_body,
        grid=grid,
        in_specs=[
            pl.BlockSpec((blk, IN_DIM), lambda i: (i, 0)),
            pl.BlockSpec((blk, 2 * D), lambda i: (i, 0)),
            pl.BlockSpec((blk, D), lambda i: (i, 0)),
            pl.BlockSpec((IN_DIM, D), lambda i: (0, 0)),
            pl.BlockSpec((D, 16), lambda i: (0, 0)),
            pl.BlockSpec((16, D), lambda i: (0, 0)),
        ],
        out_specs=[
            pl.BlockSpec((blk, D), lambda i: (i, 0)),
            pl.BlockSpec((blk, D), lambda i: (i, 0)),
            pl.BlockSpec((blk, D), lambda i: (i, 0)),
        ],
        out_shape=[
            jax.ShapeDtypeStruct((N_EDGES, D), jnp.float32),
            jax.ShapeDtypeStruct((N_EDGES, D), jnp.float32),
            jax.ShapeDtypeStruct((N_EDGES, D), jnp.float32),
        ],
    )(e, kvg, qg, We, Hmat, Rmat)


def _scatter_body(sv_hbm, sx_hbm, dst3_hbm, zeros_hbm, wv_out, zx_out,
                  didx2, rows, lsem, ssem, acc):
    cid = lax.axis_index("c")
    sid = lax.axis_index("s")
    wid = sid * NC + cid
    base = wid * EDGES_PER_W
    r0 = sid * ROWS_PER_TILE

    pltpu.sync_copy(dst3_hbm.at[wid], didx2)

    for src_hbm, out_hbm in ((sv_hbm, wv_out), (sx_hbm, zx_out)):
        pltpu.sync_copy(zeros_hbm.at[pl.ds(0, ROWS_PER_TILE)],
                        acc.at[pl.ds(r0, ROWS_PER_TILE)])

        @pl.when(sid == 0)
        def _zero_tail():
            pltpu.sync_copy(zeros_hbm.at[pl.ds(0, TAIL_ROWS)],
                            acc.at[pl.ds(TAIL_OFF, TAIL_ROWS)])

        plsc.subcore_barrier()

        def load(i, b):
            off = base + i * CH
            pltpu.async_copy(src_hbm.at[pl.ds(off, CH)], rows[b], lsem[b])

        def wait_load(i, b):
            off = base + i * CH
            pltpu.make_async_copy(src_hbm.at[pl.ds(off, CH)], rows[b],
                                  lsem[b]).wait()

        load(0, 0)

        def pbody(p, carry):
            j0 = 2 * p
            wait_load(j0, 0)
            load(j0 + 1, 1)
            pltpu.sync_copy(rows[0], acc.at[didx2.at[j0]], add=True)
            wait_load(j0 + 1, 1)
            load(j0 + 2, 0)
            pltpu.sync_copy(rows[1], acc.at[didx2.at[j0 + 1]], add=True)
            return carry

        lax.fori_loop(0, (N_CHUNKS - 1) // 2, pbody, 0)
        wait_load(N_CHUNKS - 1, 0)
        pltpu.sync_copy(rows[0], acc.at[didx2.at[N_CHUNKS - 1]], add=True)
        plsc.subcore_barrier()

        pltpu.sync_copy(acc.at[pl.ds(r0, ROWS_PER_TILE)],
                        out_hbm.at[cid, pl.ds(r0, ROWS_PER_TILE)])

        @pl.when(sid == 0)
        def _copy_tail():
            pltpu.sync_copy(acc.at[pl.ds(TAIL_OFF, TAIL_ROWS)],
                            out_hbm.at[cid, pl.ds(TAIL_OFF, TAIL_ROWS)])

        plsc.subcore_barrier()


def _scatter(sv, sx, dst, zeros):
    mesh = plsc.VectorSubcoreMesh(core_axis_name="c", subcore_axis_name="s")
    f = pl.kernel(
        _scatter_body,
        out_type=[
            jax.ShapeDtypeStruct((NC, N_NODES, D), jnp.float32),
            jax.ShapeDtypeStruct((NC, N_NODES, D), jnp.float32),
        ],
        mesh=mesh,
        scratch_types=[
            pltpu.VMEM((N_CHUNKS, CH), jnp.int32),
            [pltpu.VMEM((CH, D), jnp.float32)] * 2,
            [pltpu.SemaphoreType.DMA] * 2,
            [pltpu.SemaphoreType.DMA] * 2,
            pltpu.VMEM_SHARED((N_NODES, D), jnp.float32),
        ],
    )
    return f(sv, sx, dst.reshape(NW, N_CHUNKS, CH), zeros)


def _combine_body(wv0_ref, wv1_ref, z0_ref, z1_ref, out_ref):
    wv = wv0_ref[...] + wv1_ref[...]
    zexp = z0_ref[...] + z1_ref[...]
    out_ref[...] = wv / (zexp + 1e-6)


def _combine(wv_part, zx_part):
    blk = 1000
    grid = (N_NODES // blk,)
    return pl.pallas_call(
        _combine_body,
        grid=grid,
        in_specs=[
            pl.BlockSpec((blk, D), lambda i: (i, 0)),
            pl.BlockSpec((blk, D), lambda i: (i, 0)),
            pl.BlockSpec((blk, D), lambda i: (i, 0)),
            pl.BlockSpec((blk, D), lambda i: (i, 0)),
        ],
        out_specs=pl.BlockSpec((blk, D), lambda i: (i, 0)),
        out_shape=jax.ShapeDtypeStruct((N_NODES, D), jnp.float32),
    )(wv_part[0], wv_part[1], zx_part[0], zx_part[1])


_HMAT = np.zeros((D, 16), np.float32)
for _h in range(NUM_HEADS):
    _HMAT[_h * OUT_DIM:(_h + 1) * OUT_DIM, _h] = 1.0
_RMAT = np.zeros((16, D), np.float32)
for _h in range(NUM_HEADS):
    _RMAT[_h, _h * OUT_DIM:(_h + 1) * OUT_DIM] = 1.0


def kernel(h, e, edge_index, Wq, Wk, Wv, We):
    src = edge_index[0]
    dst = edge_index[1]
    Wkv = jnp.concatenate([Wk, Wv], axis=1)
    Hmat = jnp.asarray(_HMAT)
    Rmat = jnp.asarray(_RMAT)
    zeros = jnp.zeros((ROWS_PER_TILE, D), jnp.float32)

    kv, q = _project(h, Wkv, Wq)
    kvg, qg = _gather(kv, q, src, dst)
    e_out, sv, sx = _edge_stage(e, kvg, qg, We, Hmat, Rmat)
    wv_part, zx_part = _scatter(sv, sx, dst, zeros)
    h_out = _combine(wv_part, zx_part)
    return h_out, e_out
